# baseline (speedup 1.0000x reference)
import functools

import jax
import jax.numpy as jnp
from jax import lax
from jax.experimental import pallas as pl
from jax.experimental.pallas import tpu as pltpu
from jax.experimental.pallas import tpu_sc as plsc

N = 10000
E = 160000
D = 256
DQ = 64
NC = 2
NS = 16
L = 16
CH = 128
NCHT = 79
ECH = NS * NCHT
EPAD = ECH * CH - E
NB = 4
NP = 10240
NRT = NP // NS
DW = 16
SS = NRT // CH


def _sc_body(tbl, sq0, sq1, sq2, sq3, d2d, zeros, zeros_d,
             out_q0, out_q1, out_q2, out_q3, out_deg,
             idx_all, dst_all, rows0, rows1, ones_v, dstage_v,
             acc_s, deg_s,
             sg0, sg1, ss0, ss1, sd0, sd1):
    c = lax.axis_index("c")
    s = lax.axis_index("s")
    row0 = s * NRT
    slab0 = s * NCHT

    ones = jnp.full((L,), 1.0, dtype=jnp.float32)
    for r in range(CH):
        ones_v[r, :] = ones
    pltpu.sync_copy(d2d.at[pl.ds(slab0, NCHT)], dst_all)
    pltpu.sync_copy(zeros_d, dstage_v)
    pltpu.sync_copy(dstage_v, deg_s.at[pl.ds(row0, NRT)])

    rows_b = (rows0, rows1)
    sg = (sg0, sg1)
    ss = (ss0, ss1)
    sd = (sd0, sd1)

    def scat_start(j, b, do_deg):
        pltpu.async_copy(rows_b[b], acc_s.at[dst_all.at[j]], ss[b], add=True)
        if do_deg:
            pltpu.async_copy(ones_v, deg_s.at[dst_all.at[j]], sd[b], add=True)

    def scat_wait(b, do_deg):
        pltpu.make_async_copy(rows_b[b], acc_s.at[dst_all.at[0]], ss[b]).wait()
        if do_deg:
            pltpu.make_async_copy(ones_v, deg_s.at[dst_all.at[0]], sd[b]).wait()

    for p in range(2):
        do_deg = p == 0

        src_lo = (sq0, sq1)[p]
        src_hi = (sq2, sq3)[p]

        @pl.when(c == 0)
        def _(src=src_lo):
            pltpu.sync_copy(src.at[pl.ds(slab0, NCHT)], idx_all)

        @pl.when(c == 1)
        def _(src=src_hi):
            pltpu.sync_copy(src.at[pl.ds(slab0, NCHT)], idx_all)

        def gather_start(j, b):
            pltpu.async_copy(tbl.at[idx_all.at[j]], rows_b[b], sg[b])

        def gather_wait(b):
            pltpu.make_async_copy(tbl.at[idx_all.at[0]], rows_b[b], sg[b]).wait()

        pltpu.sync_copy(zeros, rows0)
        for k in range(SS):
            pltpu.sync_copy(rows0, acc_s.at[pl.ds(row0 + k * CH, CH)])
        plsc.subcore_barrier()

        gather_start(0, 0)

        def body(k, carry):
            a = 2 * k
            gather_wait(0)
            scat_start(a, 0, do_deg)

            @pl.when(k > 0)
            def _():
                scat_wait(1, do_deg)

            gather_start(a + 1, 1)
            gather_wait(1)
            scat_start(a + 1, 1, do_deg)
            scat_wait(0, do_deg)
            gather_start(a + 2, 0)
            return carry

        lax.fori_loop(0, (NCHT - 1) // 2, body, 0)

        gather_wait(0)
        scat_start(NCHT - 1, 0, do_deg)
        scat_wait(1, do_deg)
        scat_wait(0, do_deg)
        plsc.subcore_barrier()

        dst_lo = (out_q0, out_q1)[p]
        dst_hi = (out_q2, out_q3)[p]
        for k in range(SS):
            pltpu.sync_copy(acc_s.at[pl.ds(row0 + k * CH, CH)], rows0)

            @pl.when(c == 0)
            def _(dst=dst_lo, k=k):
                pltpu.sync_copy(rows0, dst.at[pl.ds(row0 + k * CH, CH)])

            @pl.when(c == 1)
            def _(dst=dst_hi, k=k):
                pltpu.sync_copy(rows0, dst.at[pl.ds(row0 + k * CH, CH)])

        if p == 0:
            @pl.when(c == 0)
            def _():
                pltpu.sync_copy(deg_s.at[pl.ds(row0, NRT)], dstage_v)
                pltpu.sync_copy(dstage_v, out_deg.at[pl.ds(row0, NRT)])


_sc_agg = functools.partial(
    pl.kernel,
    out_type=[
        jax.ShapeDtypeStruct((NP, DQ), jnp.float32),
        jax.ShapeDtypeStruct((NP, DQ), jnp.float32),
        jax.ShapeDtypeStruct((NP, DQ), jnp.float32),
        jax.ShapeDtypeStruct((NP, DQ), jnp.float32),
        jax.ShapeDtypeStruct((NP, DW), jnp.float32),
    ],
    mesh=plsc.VectorSubcoreMesh(core_axis_name="c", subcore_axis_name="s"),
    compiler_params=pltpu.CompilerParams(use_tc_tiling_on_sc=False),
    scratch_types=[
        pltpu.VMEM((NCHT, CH), jnp.int32),
        pltpu.VMEM((NCHT, CH), jnp.int32),
        pltpu.VMEM((CH, DQ), jnp.float32),
        pltpu.VMEM((CH, DQ), jnp.float32),
        pltpu.VMEM((CH, DW), jnp.float32),
        pltpu.VMEM((NRT, DW), jnp.float32),
        pltpu.VMEM_SHARED((NP, DQ), jnp.float32),
        pltpu.VMEM_SHARED((NP, DW), jnp.float32),
    ] + [pltpu.SemaphoreType.DMA] * 6,
)(_sc_body)


BR = 1000


def _tc_self_body(feat_b, ws, bias, out_b):
    out_b[...] = jnp.dot(feat_b[...], ws[...],
                         preferred_element_type=jnp.float32) + bias[...]


def _tc_neigh_body(self_b, n0_b, n1_b, n2_b, n3_b, deg_b,
                   wn0, wn1, wn2, wn3, out_b):
    inv = 1.0 / jnp.maximum(deg_b[...], 1.0)
    acc = self_b[...]
    acc += jnp.dot(n0_b[...] * inv, wn0[...], preferred_element_type=jnp.float32)
    acc += jnp.dot(n1_b[...] * inv, wn1[...], preferred_element_type=jnp.float32)
    acc += jnp.dot(n2_b[...] * inv, wn2[...], preferred_element_type=jnp.float32)
    acc += jnp.dot(n3_b[...] * inv, wn3[...], preferred_element_type=jnp.float32)
    out_b[...] = acc


def kernel(feat, edge_index, W_self, b_self, W_neigh, b_neigh):
    feat = feat.astype(jnp.float32)
    srcs = edge_index[0].astype(jnp.int32)
    dsts = edge_index[1].astype(jnp.int32)
    sq = [jnp.concatenate([srcs + q * N, jnp.zeros((EPAD,), jnp.int32)]
                          ).reshape(ECH, CH) for q in range(4)]
    d2d = jnp.concatenate(
        [dsts, jnp.full((EPAD,), N, jnp.int32)]).reshape(ECH, CH)
    zeros = jnp.zeros((CH, DQ), dtype=jnp.float32)
    zeros_d = jnp.zeros((NRT, DW), dtype=jnp.float32)

    tbl = jnp.concatenate(
        [feat[:, 0 * DQ:1 * DQ], feat[:, 1 * DQ:2 * DQ],
         feat[:, 2 * DQ:3 * DQ], feat[:, 3 * DQ:4 * DQ]], axis=0)
    n0, n1, n2, n3, deg16 = _sc_agg(tbl, sq[0], sq[1], sq[2], sq[3], d2d,
                                    zeros, zeros_d)
    deg = deg16[:N, :1]

    nblk = N // BR
    out_self = pl.pallas_call(
        _tc_self_body,
        grid=(nblk,),
        in_specs=[
            pl.BlockSpec((BR, D), lambda i: (i, 0)),
            pl.BlockSpec((D, D), lambda i: (0, 0)),
            pl.BlockSpec((1, D), lambda i: (0, 0)),
        ],
        out_specs=pl.BlockSpec((BR, D), lambda i: (i, 0)),
        out_shape=jax.ShapeDtypeStruct((N, D), jnp.float32),
    )(feat, W_self, (b_self + b_neigh).reshape(1, D))

    qspec = pl.BlockSpec((BR, DQ), lambda i: (i, 0))
    wspec = pl.BlockSpec((DQ, D), lambda i: (0, 0))
    out = pl.pallas_call(
        _tc_neigh_body,
        grid=(nblk,),
        in_specs=[
            pl.BlockSpec((BR, D), lambda i: (i, 0)),
            qspec, qspec, qspec, qspec,
            pl.BlockSpec((BR, 1), lambda i: (i, 0)),
            wspec, wspec, wspec, wspec,
        ],
        out_specs=pl.BlockSpec((BR, D), lambda i: (i, 0)),
        out_shape=jax.ShapeDtypeStruct((N, D), jnp.float32),
    )(out_self, n0, n1, n2, n3, deg,
      W_neigh[0 * DQ:1 * DQ], W_neigh[1 * DQ:2 * DQ],
      W_neigh[2 * DQ:3 * DQ], W_neigh[3 * DQ:4 * DQ])
    return out


# speedup vs baseline: 1.1691x; 1.1691x over previous
import functools

import jax
import jax.numpy as jnp
from jax import lax
from jax.experimental import pallas as pl
from jax.experimental.pallas import tpu as pltpu
from jax.experimental.pallas import tpu_sc as plsc

N = 10000
E = 160000
D = 256
DQ = 64
NC = 2
NS = 16
L = 16
CH = 80
NCHT = 125
ECH = NS * NCHT
EPAD = ECH * CH - E
NB = 4
NP = 10240
NRT = NP // NS
DW = 16
SS = NRT // CH


def _sc_body(tbl, sq0, sq1, sq2, sq3, d2d, zeros, zeros_d,
             out_q0, out_q1, out_q2, out_q3, out_deg,
             idx_all, dst_all, rows0, rows1, ones_v, dstage_v,
             acc_s, deg_s,
             sg0, sg1, ss0, ss1, sd0, sd1):
    c = lax.axis_index("c")
    s = lax.axis_index("s")
    row0 = s * NRT
    slab0 = s * NCHT

    ones = jnp.full((L,), 1.0, dtype=jnp.float32)
    for r in range(CH):
        ones_v[r, :] = ones
    pltpu.sync_copy(d2d.at[pl.ds(slab0, NCHT)], dst_all)
    pltpu.sync_copy(zeros_d, dstage_v)
    pltpu.sync_copy(dstage_v, deg_s.at[pl.ds(row0, NRT)])

    rows_b = (rows0, rows1)
    sg = (sg0, sg1)
    ss = (ss0, ss1)
    sd = (sd0, sd1)

    def scat_start(j, b, do_deg):
        pltpu.async_copy(rows_b[b], acc_s.at[dst_all.at[j]], ss[b], add=True)
        if do_deg:
            pltpu.async_copy(ones_v, deg_s.at[dst_all.at[j]], sd[b], add=True)

    def scat_wait(b, do_deg):
        pltpu.make_async_copy(rows_b[b], acc_s.at[dst_all.at[0]], ss[b]).wait()
        if do_deg:
            pltpu.make_async_copy(ones_v, deg_s.at[dst_all.at[0]], sd[b]).wait()

    for p in range(2):
        do_deg = p == 0

        src_lo = (sq0, sq1)[p]
        src_hi = (sq2, sq3)[p]

        @pl.when(c == 0)
        def _(src=src_lo):
            pltpu.sync_copy(src.at[pl.ds(slab0, NCHT)], idx_all)

        @pl.when(c == 1)
        def _(src=src_hi):
            pltpu.sync_copy(src.at[pl.ds(slab0, NCHT)], idx_all)

        def gather_start(j, b):
            pltpu.async_copy(tbl.at[idx_all.at[j]], rows_b[b], sg[b])

        def gather_wait(b):
            pltpu.make_async_copy(tbl.at[idx_all.at[0]], rows_b[b], sg[b]).wait()

        pltpu.sync_copy(zeros, rows0)
        for k in range(SS):
            pltpu.sync_copy(rows0, acc_s.at[pl.ds(row0 + k * CH, CH)])
        plsc.subcore_barrier()

        gather_start(0, 0)

        def body(k, carry):
            a = 2 * k
            gather_wait(0)
            scat_start(a, 0, do_deg)

            @pl.when(k > 0)
            def _():
                scat_wait(1, do_deg)

            gather_start(a + 1, 1)
            gather_wait(1)
            scat_start(a + 1, 1, do_deg)
            scat_wait(0, do_deg)
            gather_start(a + 2, 0)
            return carry

        lax.fori_loop(0, (NCHT - 1) // 2, body, 0)

        gather_wait(0)
        scat_start(NCHT - 1, 0, do_deg)
        scat_wait(1, do_deg)
        scat_wait(0, do_deg)
        plsc.subcore_barrier()

        dst_lo = (out_q0, out_q1)[p]
        dst_hi = (out_q2, out_q3)[p]
        for k in range(SS):
            pltpu.sync_copy(acc_s.at[pl.ds(row0 + k * CH, CH)], rows0)

            @pl.when(c == 0)
            def _(dst=dst_lo, k=k):
                pltpu.sync_copy(rows0, dst.at[pl.ds(row0 + k * CH, CH)])

            @pl.when(c == 1)
            def _(dst=dst_hi, k=k):
                pltpu.sync_copy(rows0, dst.at[pl.ds(row0 + k * CH, CH)])

        if p == 0:
            @pl.when(c == 0)
            def _():
                pltpu.sync_copy(deg_s.at[pl.ds(row0, NRT)], dstage_v)
                pltpu.sync_copy(dstage_v, out_deg.at[pl.ds(row0, NRT)])


_sc_agg = functools.partial(
    pl.kernel,
    out_type=[
        jax.ShapeDtypeStruct((NP, DQ), jnp.float32),
        jax.ShapeDtypeStruct((NP, DQ), jnp.float32),
        jax.ShapeDtypeStruct((NP, DQ), jnp.float32),
        jax.ShapeDtypeStruct((NP, DQ), jnp.float32),
        jax.ShapeDtypeStruct((NP, DW), jnp.float32),
    ],
    mesh=plsc.VectorSubcoreMesh(core_axis_name="c", subcore_axis_name="s"),
    compiler_params=pltpu.CompilerParams(use_tc_tiling_on_sc=False),
    scratch_types=[
        pltpu.VMEM((NCHT, CH), jnp.int32),
        pltpu.VMEM((NCHT, CH), jnp.int32),
        pltpu.VMEM((CH, DQ), jnp.float32),
        pltpu.VMEM((CH, DQ), jnp.float32),
        pltpu.VMEM((CH, DW), jnp.float32),
        pltpu.VMEM((NRT, DW), jnp.float32),
        pltpu.VMEM_SHARED((NP, DQ), jnp.float32),
        pltpu.VMEM_SHARED((NP, DW), jnp.float32),
    ] + [pltpu.SemaphoreType.DMA] * 6,
)(_sc_body)


BR = 1000


def _tc_self_body(feat_b, ws, bias, out_b):
    out_b[...] = jnp.dot(feat_b[...], ws[...],
                         preferred_element_type=jnp.float32) + bias[...]


def _tc_neigh_body(self_b, n0_b, n1_b, n2_b, n3_b, deg_b,
                   wn0, wn1, wn2, wn3, out_b):
    inv = 1.0 / jnp.maximum(deg_b[...], 1.0)
    acc = self_b[...]
    acc += jnp.dot(n0_b[...] * inv, wn0[...], preferred_element_type=jnp.float32)
    acc += jnp.dot(n1_b[...] * inv, wn1[...], preferred_element_type=jnp.float32)
    acc += jnp.dot(n2_b[...] * inv, wn2[...], preferred_element_type=jnp.float32)
    acc += jnp.dot(n3_b[...] * inv, wn3[...], preferred_element_type=jnp.float32)
    out_b[...] = acc


def kernel(feat, edge_index, W_self, b_self, W_neigh, b_neigh):
    feat = feat.astype(jnp.float32)
    srcs = edge_index[0].astype(jnp.int32)
    dsts = edge_index[1].astype(jnp.int32)
    sq = [jnp.concatenate([srcs + q * N, jnp.zeros((EPAD,), jnp.int32)]
                          ).reshape(ECH, CH) for q in range(4)]
    d2d = jnp.concatenate(
        [dsts, jnp.full((EPAD,), N, jnp.int32)]).reshape(ECH, CH)
    zeros = jnp.zeros((CH, DQ), dtype=jnp.float32)
    zeros_d = jnp.zeros((NRT, DW), dtype=jnp.float32)

    tbl = jnp.concatenate(
        [feat[:, 0 * DQ:1 * DQ], feat[:, 1 * DQ:2 * DQ],
         feat[:, 2 * DQ:3 * DQ], feat[:, 3 * DQ:4 * DQ]], axis=0)
    n0, n1, n2, n3, deg16 = _sc_agg(tbl, sq[0], sq[1], sq[2], sq[3], d2d,
                                    zeros, zeros_d)
    deg = deg16[:N, :1]

    nblk = N // BR
    out_self = pl.pallas_call(
        _tc_self_body,
        grid=(nblk,),
        in_specs=[
            pl.BlockSpec((BR, D), lambda i: (i, 0)),
            pl.BlockSpec((D, D), lambda i: (0, 0)),
            pl.BlockSpec((1, D), lambda i: (0, 0)),
        ],
        out_specs=pl.BlockSpec((BR, D), lambda i: (i, 0)),
        out_shape=jax.ShapeDtypeStruct((N, D), jnp.float32),
    )(feat, W_self, (b_self + b_neigh).reshape(1, D))

    qspec = pl.BlockSpec((BR, DQ), lambda i: (i, 0))
    wspec = pl.BlockSpec((DQ, D), lambda i: (0, 0))
    out = pl.pallas_call(
        _tc_neigh_body,
        grid=(nblk,),
        in_specs=[
            pl.BlockSpec((BR, D), lambda i: (i, 0)),
            qspec, qspec, qspec, qspec,
            pl.BlockSpec((BR, 1), lambda i: (i, 0)),
            wspec, wspec, wspec, wspec,
        ],
        out_specs=pl.BlockSpec((BR, D), lambda i: (i, 0)),
        out_shape=jax.ShapeDtypeStruct((N, D), jnp.float32),
    )(out_self, n0, n1, n2, n3, deg,
      W_neigh[0 * DQ:1 * DQ], W_neigh[1 * DQ:2 * DQ],
      W_neigh[2 * DQ:3 * DQ], W_neigh[3 * DQ:4 * DQ])
    return out
